# Initial kernel scaffold; baseline (speedup 1.0000x reference)
import jax
import jax.numpy as jnp
from jax.experimental import pallas as pl


def kernel(x, tables, W1, b1, W2, b2):
    raise NotImplementedError("write your pallas kernel here")


# speedup vs baseline: 8.0968x; 8.0968x over previous
import functools

import jax
import jax.numpy as jnp
from jax import lax
from jax.experimental import pallas as pl
from jax.experimental.pallas import tpu as pltpu
from jax.experimental.pallas import tpu_sc as plsc

B = 16384
F = 26
V = 100000
D = 32
H = 256
OUT = 128

NC = 2
NS = 16
NW = NC * NS
ROWS = B * F
ROWS_PER_W = ROWS // NW
CHUNK = 1024
NCHUNK = ROWS_PER_W // CHUNK

_sc_mesh = plsc.VectorSubcoreMesh(core_axis_name="c", subcore_axis_name="s")


@functools.partial(
    pl.kernel,
    mesh=_sc_mesh,
    out_type=jax.ShapeDtypeStruct((ROWS, D), jnp.float32),
    scratch_types=[
        pltpu.VMEM((ROWS_PER_W,), jnp.int32),
        pltpu.VMEM((CHUNK, D), jnp.float32),
        pltpu.SemaphoreType.DMA,
    ],
    compiler_params=pltpu.CompilerParams(use_tc_tiling_on_sc=False),
)
def _sc_gather(table_hbm, idx_hbm, out_hbm, idx_v, rows_v, sem):
    wid = lax.axis_index("s") * NC + lax.axis_index("c")
    base = wid * ROWS_PER_W
    pltpu.sync_copy(idx_hbm.at[pl.ds(base, ROWS_PER_W)], idx_v)
    for k in range(NCHUNK):
        cp = pltpu.async_copy(
            table_hbm.at[idx_v.at[pl.ds(k * CHUNK, CHUNK)]], rows_v, sem)
        cp.wait()
        pltpu.sync_copy(rows_v, out_hbm.at[pl.ds(base + k * CHUNK, CHUNK)])


BLK = 1024


def _mlp_body(f_ref, w1_ref, b1_ref, w2_ref, b2_ref, o_ref):
    h = jnp.dot(f_ref[...], w1_ref[...], preferred_element_type=jnp.float32)
    h = jnp.maximum(h + b1_ref[...], 0.0)
    o_ref[...] = (
        jnp.dot(h, w2_ref[...], preferred_element_type=jnp.float32) + b2_ref[...])


def _mlp(flat, W1, b1, W2, b2):
    return pl.pallas_call(
        _mlp_body,
        grid=(B // BLK,),
        in_specs=[
            pl.BlockSpec((BLK, F * D), lambda i: (i, 0)),
            pl.BlockSpec((F * D, H), lambda i: (0, 0)),
            pl.BlockSpec((1, H), lambda i: (0, 0)),
            pl.BlockSpec((H, OUT), lambda i: (0, 0)),
            pl.BlockSpec((1, OUT), lambda i: (0, 0)),
        ],
        out_specs=pl.BlockSpec((BLK, OUT), lambda i: (i, 0)),
        out_shape=jax.ShapeDtypeStruct((B, OUT), jnp.float32),
    )(flat, W1, b1.reshape(1, H), W2, b2.reshape(1, OUT))


def kernel(x, tables, W1, b1, W2, b2):
    xi = x.astype(jnp.int32)
    idx = (xi + jnp.arange(F, dtype=jnp.int32)[None, :] * V).reshape(ROWS)
    table2d = tables.reshape(F * V, D)
    rows = _sc_gather(table2d, idx)
    flat = rows.reshape(B, F * D)
    return _mlp(flat, W1, b1, W2, b2)


# speedup vs baseline: 16.4180x; 2.0277x over previous
import functools

import jax
import jax.numpy as jnp
from jax import lax
from jax.experimental import pallas as pl
from jax.experimental.pallas import tpu as pltpu
from jax.experimental.pallas import tpu_sc as plsc

B = 16384
F = 26
V = 100000
D = 32
H = 256
OUT = 128

ROWS = F * D
HALF = B // 2

_sc_mesh = plsc.VectorSubcoreMesh(core_axis_name="c", subcore_axis_name="s")


@functools.partial(
    pl.kernel,
    mesh=_sc_mesh,
    out_type=jax.ShapeDtypeStruct((ROWS, B), jnp.float32),
    scratch_types=[
        pltpu.VMEM((V,), jnp.float32),
        pltpu.VMEM((HALF,), jnp.int32),
        pltpu.VMEM((HALF,), jnp.float32),
        pltpu.SemaphoreType.DMA,
    ],
    compiler_params=pltpu.CompilerParams(needs_layout_passes=False),
)
def _sc_gather(t2_hbm, xt_hbm, g_hbm, row_v, idx_v, out_v, sem):
    w = lax.axis_index("s") * 2 + lax.axis_index("c")

    @pl.when(w < F)
    def _body():
        for half in range(2):
            pltpu.sync_copy(xt_hbm.at[w, pl.ds(half * HALF, HALF)], idx_v)
            for d in range(D):
                r = w * D + d
                pltpu.sync_copy(t2_hbm.at[r, :], row_v)

                def gather_step(j, carry):
                    v16 = idx_v[pl.ds(j * 16, 16)]
                    out_v[pl.ds(j * 16, 16)] = plsc.load_gather(row_v, [v16])
                    return carry

                lax.fori_loop(0, HALF // 16, gather_step, 0, unroll=8)
                pltpu.sync_copy(out_v, g_hbm.at[r, pl.ds(half * HALF, HALF)])


BLK = 2048


def _mlp_body(g_ref, w1_ref, b1_ref, w2_ref, b2_ref, o_ref):
    t = lax.dot_general(w1_ref[...], g_ref[...], (((0,), (0,)), ((), ())),
                        preferred_element_type=jnp.float32)
    h = jnp.maximum(t + b1_ref[...], 0.0)
    o_ref[...] = lax.dot_general(h, w2_ref[...], (((0,), (0,)), ((), ())),
                                 preferred_element_type=jnp.float32) + b2_ref[...]


def _mlp_t(g, W1, b1, W2, b2):
    return pl.pallas_call(
        _mlp_body,
        grid=(B // BLK,),
        in_specs=[
            pl.BlockSpec((ROWS, BLK), lambda i: (0, i)),
            pl.BlockSpec((ROWS, H), lambda i: (0, 0)),
            pl.BlockSpec((H, 1), lambda i: (0, 0)),
            pl.BlockSpec((H, OUT), lambda i: (0, 0)),
            pl.BlockSpec((1, OUT), lambda i: (0, 0)),
        ],
        out_specs=pl.BlockSpec((BLK, OUT), lambda i: (i, 0)),
        out_shape=jax.ShapeDtypeStruct((B, OUT), jnp.float32),
    )(g, W1, b1.reshape(H, 1), W2, b2.reshape(1, OUT))


def kernel(x, tables, W1, b1, W2, b2):
    t2 = jnp.swapaxes(tables, 1, 2).reshape(ROWS, V)
    xt = jnp.swapaxes(x, 0, 1).astype(jnp.int32)
    g = _sc_gather(t2, xt)
    return _mlp_t(g, W1, b1, W2, b2)


# speedup vs baseline: 24.7109x; 1.5051x over previous
import functools

import jax
import jax.numpy as jnp
from jax import lax
from jax.experimental import pallas as pl
from jax.experimental.pallas import tpu as pltpu
from jax.experimental.pallas import tpu_sc as plsc

B = 16384
F = 26
V = 100000
D = 32
H = 256
OUT = 128

ROWS = F * D
RPW = ROWS // 32
QTR = B // 4

_sc_mesh = plsc.VectorSubcoreMesh(core_axis_name="c", subcore_axis_name="s")


@functools.partial(
    pl.kernel,
    mesh=_sc_mesh,
    out_type=jax.ShapeDtypeStruct((ROWS, B), jnp.float32),
    scratch_types=[
        pltpu.VMEM((V,), jnp.float32),
        pltpu.VMEM((B,), jnp.int32),
        pltpu.VMEM((2, QTR), jnp.float32),
        pltpu.SemaphoreType.DMA,
        pltpu.SemaphoreType.DMA,
    ],
    compiler_params=pltpu.CompilerParams(needs_layout_passes=False),
)
def _sc_gather(t2_hbm, xt_hbm, g_hbm, row_v, idx_v, out_v, sem0, sem1):
    w = lax.axis_index("s") * 2 + lax.axis_index("c")
    r0 = w * RPW
    sems = (sem0, sem1)
    pending = [None, None]
    for i in range(RPW):
        r = r0 + i
        f = r // D

        @pl.when(jnp.logical_or(i == 0, r % D == 0))
        def _load_idx(f=f):
            pltpu.sync_copy(xt_hbm.at[f, :], idx_v)

        pltpu.sync_copy(t2_hbm.at[r, :], row_v)
        for q in range(4):
            b = q % 2
            if pending[b] is not None:
                pending[b].wait()

            def gather_step(j, carry, q=q, b=b):
                v16 = idx_v[pl.ds(q * QTR + j * 16, 16)]
                out_v[b, pl.ds(j * 16, 16)] = plsc.load_gather(row_v, [v16])
                return carry

            lax.fori_loop(0, QTR // 16, gather_step, 0, unroll=8)
            pending[b] = pltpu.async_copy(
                out_v.at[b], g_hbm.at[r, pl.ds(q * QTR, QTR)], sems[b])
    for b in range(2):
        if pending[b] is not None:
            pending[b].wait()


BLK = 2048


def _mlp_body(g_ref, w1_ref, b1_ref, w2_ref, b2_ref, o_ref):
    t = lax.dot_general(w1_ref[...], g_ref[...], (((0,), (0,)), ((), ())),
                        preferred_element_type=jnp.float32)
    h = jnp.maximum(t + b1_ref[...], 0.0)
    o_ref[...] = lax.dot_general(h, w2_ref[...], (((0,), (0,)), ((), ())),
                                 preferred_element_type=jnp.float32) + b2_ref[...]


def _mlp_t(g, W1, b1, W2, b2):
    return pl.pallas_call(
        _mlp_body,
        grid=(B // BLK,),
        in_specs=[
            pl.BlockSpec((ROWS, BLK), lambda i: (0, i)),
            pl.BlockSpec((ROWS, H), lambda i: (0, 0)),
            pl.BlockSpec((H, 1), lambda i: (0, 0)),
            pl.BlockSpec((H, OUT), lambda i: (0, 0)),
            pl.BlockSpec((1, OUT), lambda i: (0, 0)),
        ],
        out_specs=pl.BlockSpec((BLK, OUT), lambda i: (i, 0)),
        out_shape=jax.ShapeDtypeStruct((B, OUT), jnp.float32),
    )(g, W1, b1.reshape(H, 1), W2, b2.reshape(1, OUT))


def kernel(x, tables, W1, b1, W2, b2):
    t2 = jnp.swapaxes(tables, 1, 2).reshape(ROWS, V)
    xt = jnp.swapaxes(x, 0, 1).astype(jnp.int32)
    g = _sc_gather(t2, xt)
    return _mlp_t(g, W1, b1, W2, b2)


# speedup vs baseline: 45.1881x; 1.8287x over previous
import functools

import jax
import jax.numpy as jnp
from jax import lax
from jax.experimental import pallas as pl
from jax.experimental.pallas import tpu as pltpu
from jax.experimental.pallas import tpu_sc as plsc

B = 16384
F = 26
V = 100000
D = 32
H = 256
OUT = 128

ROWS = F * D
RPW = ROWS // 32
QTR = B // 4

_sc_mesh = plsc.VectorSubcoreMesh(core_axis_name="c", subcore_axis_name="s")


@functools.partial(
    pl.kernel,
    mesh=_sc_mesh,
    out_type=jax.ShapeDtypeStruct((ROWS, B), jnp.float32),
    scratch_types=[
        pltpu.VMEM((V,), jnp.float32),
        pltpu.VMEM((B,), jnp.int32),
        pltpu.VMEM((2, QTR), jnp.float32),
        pltpu.SemaphoreType.DMA,
        pltpu.SemaphoreType.DMA,
    ],
    compiler_params=pltpu.CompilerParams(needs_layout_passes=False),
)
def _sc_gather(t2_hbm, xt_hbm, g_hbm, row_v, idx_v, out_v, sem0, sem1):
    w = lax.axis_index("s") * 2 + lax.axis_index("c")
    r0 = w * RPW
    sems = (sem0, sem1)
    pending = [None, None]
    for i in range(RPW):
        r = r0 + i
        f = r // D

        @pl.when(jnp.logical_or(i == 0, r % D == 0))
        def _load_idx(f=f):
            pltpu.sync_copy(xt_hbm.at[f, :], idx_v)

        pltpu.sync_copy(t2_hbm.at[r, :], row_v)
        for q in range(4):
            b = q % 2
            if pending[b] is not None:
                pending[b].wait()

            @plsc.parallel_loop(0, QTR, 16, unroll=8)
            def gather_step(j, q=q, b=b):
                v16 = idx_v[pl.ds(q * QTR + j, 16)]
                out_v[b, pl.ds(j, 16)] = plsc.load_gather(row_v, [v16])
            pending[b] = pltpu.async_copy(
                out_v.at[b], g_hbm.at[r, pl.ds(q * QTR, QTR)], sems[b])
    for b in range(2):
        if pending[b] is not None:
            pending[b].wait()


BLK = 2048


def _mlp_body(g_ref, w1_ref, b1_ref, w2_ref, b2_ref, o_ref):
    t = lax.dot_general(w1_ref[...], g_ref[...], (((0,), (0,)), ((), ())),
                        preferred_element_type=jnp.float32)
    h = jnp.maximum(t + b1_ref[...], 0.0)
    o_ref[...] = lax.dot_general(h, w2_ref[...], (((0,), (0,)), ((), ())),
                                 preferred_element_type=jnp.float32) + b2_ref[...]


def _mlp_t(g, W1, b1, W2, b2):
    return pl.pallas_call(
        _mlp_body,
        grid=(B // BLK,),
        in_specs=[
            pl.BlockSpec((ROWS, BLK), lambda i: (0, i)),
            pl.BlockSpec((ROWS, H), lambda i: (0, 0)),
            pl.BlockSpec((H, 1), lambda i: (0, 0)),
            pl.BlockSpec((H, OUT), lambda i: (0, 0)),
            pl.BlockSpec((1, OUT), lambda i: (0, 0)),
        ],
        out_specs=pl.BlockSpec((BLK, OUT), lambda i: (i, 0)),
        out_shape=jax.ShapeDtypeStruct((B, OUT), jnp.float32),
    )(g, W1, b1.reshape(H, 1), W2, b2.reshape(1, OUT))


def kernel(x, tables, W1, b1, W2, b2):
    t2 = jnp.swapaxes(tables, 1, 2).reshape(ROWS, V)
    xt = jnp.swapaxes(x, 0, 1).astype(jnp.int32)
    g = _sc_gather(t2, xt)
    return _mlp_t(g, W1, b1, W2, b2)


# speedup vs baseline: 45.2731x; 1.0019x over previous
import functools

import jax
import jax.numpy as jnp
from jax import lax
from jax.experimental import pallas as pl
from jax.experimental.pallas import tpu as pltpu
from jax.experimental.pallas import tpu_sc as plsc

B = 16384
F = 26
V = 100000
D = 32
H = 256
OUT = 128

ROWS = F * D
RPW = ROWS // 32
QTR = B // 4

_sc_mesh = plsc.VectorSubcoreMesh(core_axis_name="c", subcore_axis_name="s")


@functools.partial(
    pl.kernel,
    mesh=_sc_mesh,
    out_type=jax.ShapeDtypeStruct((ROWS, B), jnp.float32),
    scratch_types=[
        pltpu.VMEM((V,), jnp.float32),
        pltpu.VMEM((B,), jnp.int32),
        pltpu.VMEM((2, QTR), jnp.float32),
        pltpu.SemaphoreType.DMA,
        pltpu.SemaphoreType.DMA,
        pltpu.SemaphoreType.DMA,
        pltpu.SemaphoreType.DMA,
    ],
    compiler_params=pltpu.CompilerParams(needs_layout_passes=False),
)
def _sc_gather(t2_hbm, xt_hbm, g_hbm, row_v, idx_v, out_v, sem0, sem1,
               semr0, semr1):
    w = lax.axis_index("s") * 2 + lax.axis_index("c")
    r0 = w * RPW
    sems = (sem0, sem1)
    pending = [None, None]
    for i in range(RPW):
        r = r0 + i
        f = r // D

        @pl.when(jnp.logical_or(i == 0, r % D == 0))
        def _load_idx(f=f):
            pltpu.sync_copy(xt_hbm.at[f, :], idx_v)

        pltpu.sync_copy(t2_hbm.at[r, :], row_v)
        for q in range(4):
            b = q % 2
            if pending[b] is not None:
                pending[b].wait()

            @plsc.parallel_loop(0, QTR, 16, unroll=8)
            def gather_step(j, q=q, b=b):
                v16 = idx_v[pl.ds(q * QTR + j, 16)]
                out_v[b, pl.ds(j, 16)] = plsc.load_gather(row_v, [v16])
            pending[b] = pltpu.async_copy(
                out_v.at[b], g_hbm.at[r, pl.ds(q * QTR, QTR)], sems[b])
    for b in range(2):
        if pending[b] is not None:
            pending[b].wait()


BLK = 2048


def _mlp_body(g_ref, w1_ref, b1_ref, w2_ref, b2_ref, o_ref):
    t = lax.dot_general(w1_ref[...], g_ref[...], (((0,), (0,)), ((), ())),
                        preferred_element_type=jnp.float32)
    h = jnp.maximum(t + b1_ref[...], 0.0)
    o_ref[...] = lax.dot_general(h, w2_ref[...], (((0,), (0,)), ((), ())),
                                 preferred_element_type=jnp.float32) + b2_ref[...]


def _mlp_t(g, W1, b1, W2, b2):
    return pl.pallas_call(
        _mlp_body,
        grid=(B // BLK,),
        in_specs=[
            pl.BlockSpec((ROWS, BLK), lambda i: (0, i)),
            pl.BlockSpec((ROWS, H), lambda i: (0, 0)),
            pl.BlockSpec((H, 1), lambda i: (0, 0)),
            pl.BlockSpec((H, OUT), lambda i: (0, 0)),
            pl.BlockSpec((1, OUT), lambda i: (0, 0)),
        ],
        out_specs=pl.BlockSpec((BLK, OUT), lambda i: (i, 0)),
        out_shape=jax.ShapeDtypeStruct((B, OUT), jnp.float32),
    )(g, W1, b1.reshape(H, 1), W2, b2.reshape(1, OUT))


def kernel(x, tables, W1, b1, W2, b2):
    t2 = jnp.swapaxes(tables, 1, 2).reshape(ROWS, V)
    xt = jnp.swapaxes(x, 0, 1).astype(jnp.int32)
    g = _sc_gather(t2, xt)
    return _mlp_t(g, W1, b1, W2, b2)
